# baseline (device time: 114493 ns/iter reference)
import jax
import jax.numpy as jnp
from jax import lax
from jax.experimental import pallas as pl
from jax.experimental.pallas import tpu as pltpu

N_DEV = 16
N_TOK = 2048
D_IN = 512
D_OUT = 1024
N_EXP = 128
E_LOCAL = N_EXP // N_DEV
CAP = 12
CHUNK = E_LOCAL * CAP
TOK_PER = N_TOK // N_DEV


def _expert_mm_allgather(xe, expert_W):

    def body(xe_ref, ew_ref, out_ref, send_sems, recv_sems):
        my = lax.axis_index("i")
        left = lax.rem(my - 1 + N_DEV, N_DEV)
        right = lax.rem(my + 1, N_DEV)

        barrier_sem = pltpu.get_barrier_semaphore()
        for nbr in (left, right):
            pl.semaphore_signal(
                barrier_sem,
                inc=1,
                device_id=(nbr,),
                device_id_type=pl.DeviceIdType.MESH,
            )
        pl.semaphore_wait(barrier_sem, 2)

        xe3 = xe_ref[...].reshape(E_LOCAL, CAP, D_IN)
        y = lax.dot_general(
            xe3,
            ew_ref[...],
            dimension_numbers=(((2,), (1,)), ((0,), (0,))),
            preferred_element_type=jnp.float32,
        ).reshape(CHUNK, D_OUT)
        out_ref[pl.ds(my * CHUNK, CHUNK), :] = y

        for h in range(N_DEV - 1):
            o_send = lax.rem(my - h + N_DEV, N_DEV)
            o_recv = lax.rem(my - h - 1 + N_DEV, N_DEV)
            send = pltpu.make_async_remote_copy(
                src_ref=out_ref.at[pl.ds(o_send * CHUNK, CHUNK)],
                dst_ref=out_ref.at[pl.ds(o_send * CHUNK, CHUNK)],
                send_sem=send_sems.at[h],
                recv_sem=recv_sems.at[h],
                device_id=(right,),
                device_id_type=pl.DeviceIdType.MESH,
            )
            send.start()
            recv = pltpu.make_async_remote_copy(
                src_ref=out_ref.at[pl.ds(o_recv * CHUNK, CHUNK)],
                dst_ref=out_ref.at[pl.ds(o_recv * CHUNK, CHUNK)],
                send_sem=send_sems.at[h],
                recv_sem=recv_sems.at[h],
                device_id=(left,),
                device_id_type=pl.DeviceIdType.MESH,
            )
            recv.wait_recv()
            send.wait_send()

    return pl.pallas_call(
        body,
        out_shape=jax.ShapeDtypeStruct((N_DEV * CHUNK, D_OUT), jnp.float32),
        in_specs=[
            pl.BlockSpec(memory_space=pltpu.VMEM),
            pl.BlockSpec(memory_space=pltpu.VMEM),
        ],
        out_specs=pl.BlockSpec(memory_space=pltpu.VMEM),
        scratch_shapes=[
            pltpu.SemaphoreType.DMA((N_DEV - 1,)),
            pltpu.SemaphoreType.DMA((N_DEV - 1,)),
        ],
        compiler_params=pltpu.CompilerParams(collective_id=0),
    )(xe, expert_W)


def kernel(x, router_W, route_idx, expert_W):
    del router_W
    my = lax.axis_index("i")

    e = route_idx[:, 0].astype(jnp.int32)
    onehot = (e[:, None] == jnp.arange(N_EXP, dtype=jnp.int32)[None, :]).astype(
        jnp.int32
    )
    rank = jnp.sum(jnp.cumsum(onehot, axis=0) * onehot, axis=1) - 1
    kept = rank < CAP
    slot = jnp.where(kept, e * CAP + rank, N_EXP * CAP)

    tok_for_slot = (
        jnp.zeros(N_EXP * CAP + 1, jnp.int32)
        .at[slot]
        .set(jnp.arange(N_TOK, dtype=jnp.int32))
    )
    my_slots = lax.dynamic_slice(tok_for_slot, (my * CHUNK,), (CHUNK,))
    xe = jnp.take(x, my_slots, axis=0)

    gathered = _expert_mm_allgather(xe, expert_W)

    row0 = my * TOK_PER
    out_slot = lax.dynamic_slice(slot, (row0,), (TOK_PER,))
    out_kept = lax.dynamic_slice(kept, (row0,), (TOK_PER,))
    out = jnp.take(gathered, jnp.minimum(out_slot, N_EXP * CAP - 1), axis=0)
    return jnp.where(out_kept[:, None], out, 0.0)


# device time: 47208 ns/iter; 2.4253x vs baseline; 2.4253x over previous
import functools

import jax
import jax.numpy as jnp
from jax import lax
from jax.experimental import pallas as pl
from jax.experimental.pallas import tpu as pltpu

N_DEV = 16
N_TOK = 2048
D_IN = 512
D_OUT = 1024
N_EXP = 128
E_LOCAL = N_EXP // N_DEV
CAP = 12
CHUNK = E_LOCAL * CAP
TOK_PER = N_TOK // N_DEV


def _moe_a2a(xe, expert_W, send_dst, send_row, send_valid, loc_slot, counts):

    def body(
        xe_ref,
        ew_ref,
        send_dst_ref,
        send_row_ref,
        send_valid_ref,
        loc_slot_ref,
        counts_ref,
        out_ref,
        y_ref,
        send_sem,
        recv_sem,
    ):
        me = lax.axis_index("i")

        out_ref[...] = jnp.zeros((TOK_PER, D_OUT), jnp.float32)

        barrier_sem = pltpu.get_barrier_semaphore()
        for d in range(N_DEV):
            @pl.when(d != me)
            def _():
                pl.semaphore_signal(
                    barrier_sem,
                    inc=1,
                    device_id=(d,),
                    device_id_type=pl.DeviceIdType.MESH,
                )
        pl.semaphore_wait(barrier_sem, N_DEV - 1)

        xe3 = xe_ref[...].reshape(E_LOCAL, CAP, D_IN)
        y_ref[...] = lax.dot_general(
            xe3,
            ew_ref[...],
            dimension_numbers=(((2,), (1,)), ((0,), (0,))),
            preferred_element_type=jnp.float32,
        ).reshape(CHUNK, D_OUT)

        def send_one(s, _):
            @pl.when(send_valid_ref[s] == 1)
            def _():
                rdma = pltpu.make_async_remote_copy(
                    src_ref=y_ref.at[pl.ds(s, 1)],
                    dst_ref=out_ref.at[pl.ds(send_row_ref[s], 1)],
                    send_sem=send_sem,
                    recv_sem=recv_sem,
                    device_id=(send_dst_ref[s],),
                    device_id_type=pl.DeviceIdType.MESH,
                )
                rdma.start()
            return 0

        lax.fori_loop(0, CHUNK, send_one, 0)

        def copy_one(r, _):
            s = loc_slot_ref[r]

            @pl.when(s >= 0)
            def _():
                out_ref[pl.ds(r, 1), :] = y_ref[pl.ds(s, 1), :]

            return 0

        lax.fori_loop(0, TOK_PER, copy_one, 0)

        def wait_recv_one(k, _):
            recv = pltpu.make_async_remote_copy(
                src_ref=y_ref.at[pl.ds(0, 1)],
                dst_ref=out_ref.at[pl.ds(0, 1)],
                send_sem=send_sem,
                recv_sem=recv_sem,
                device_id=(me,),
                device_id_type=pl.DeviceIdType.MESH,
            )
            recv.wait_recv()
            return 0

        lax.fori_loop(0, counts_ref[1], wait_recv_one, 0)

        def wait_send_one(k, _):
            snd = pltpu.make_async_remote_copy(
                src_ref=y_ref.at[pl.ds(0, 1)],
                dst_ref=out_ref.at[pl.ds(0, 1)],
                send_sem=send_sem,
                recv_sem=recv_sem,
                device_id=(me,),
                device_id_type=pl.DeviceIdType.MESH,
            )
            snd.wait_send()
            return 0

        lax.fori_loop(0, counts_ref[0], wait_send_one, 0)

        @functools.partial(
            pl.run_scoped, second_barrier=pltpu.SemaphoreType.REGULAR
        )
        def _(second_barrier):
            for d in range(N_DEV):
                @pl.when(d != me)
                def _():
                    pl.semaphore_signal(
                        second_barrier,
                        inc=1,
                        device_id=(d,),
                        device_id_type=pl.DeviceIdType.MESH,
                    )
            pl.semaphore_wait(second_barrier, N_DEV - 1)

    smem = pl.BlockSpec(memory_space=pltpu.SMEM)
    vmem = pl.BlockSpec(memory_space=pltpu.VMEM)
    return pl.pallas_call(
        body,
        out_shape=jax.ShapeDtypeStruct((TOK_PER, D_OUT), jnp.float32),
        in_specs=[vmem, vmem, smem, smem, smem, smem, smem],
        out_specs=vmem,
        scratch_shapes=[
            pltpu.VMEM((CHUNK, D_OUT), jnp.float32),
            pltpu.SemaphoreType.DMA,
            pltpu.SemaphoreType.DMA,
        ],
        compiler_params=pltpu.CompilerParams(collective_id=0),
    )(xe, expert_W, send_dst, send_row, send_valid, loc_slot, counts)


def kernel(x, router_W, route_idx, expert_W):
    del router_W
    my = lax.axis_index("i")

    e = route_idx[:, 0].astype(jnp.int32)
    onehot = (e[:, None] == jnp.arange(N_EXP, dtype=jnp.int32)[None, :]).astype(
        jnp.int32
    )
    rank = jnp.sum(jnp.cumsum(onehot, axis=0) * onehot, axis=1) - 1
    kept = rank < CAP
    slot = jnp.where(kept, e * CAP + rank, N_EXP * CAP)

    arange_tok = jnp.arange(N_TOK, dtype=jnp.int32)
    tok_for_slot = jnp.zeros(N_EXP * CAP + 1, jnp.int32).at[slot].set(arange_tok)
    slot_used = jnp.zeros(N_EXP * CAP + 1, jnp.int32).at[slot].set(1)

    my_toks = lax.dynamic_slice(tok_for_slot, (my * CHUNK,), (CHUNK,))
    my_used = lax.dynamic_slice(slot_used, (my * CHUNK,), (CHUNK,))
    send_dst = my_toks // TOK_PER
    send_row = my_toks % TOK_PER
    send_valid = my_used * (send_dst != my).astype(jnp.int32)
    n_send = jnp.sum(send_valid)

    row0 = my * TOK_PER
    my_e = lax.dynamic_slice(e, (row0,), (TOK_PER,))
    my_rank = lax.dynamic_slice(rank, (row0,), (TOK_PER,))
    my_kept = lax.dynamic_slice(kept, (row0,), (TOK_PER,))
    owner = my_e // E_LOCAL
    local = my_kept & (owner == my)
    loc_slot = jnp.where(
        local, (my_e % E_LOCAL) * CAP + my_rank, jnp.int32(-1)
    ).astype(jnp.int32)
    n_recv = jnp.sum(my_kept & (owner != my)).astype(jnp.int32)
    counts = jnp.stack([n_send.astype(jnp.int32), n_recv])

    xe = jnp.take(x, my_toks, axis=0)

    return _moe_a2a(
        xe,
        expert_W,
        send_dst.astype(jnp.int32),
        send_row.astype(jnp.int32),
        send_valid.astype(jnp.int32),
        loc_slot,
        counts,
    )


# device time: 31211 ns/iter; 3.6684x vs baseline; 1.5125x over previous
import functools

import jax
import jax.numpy as jnp
from jax import lax
from jax.experimental import pallas as pl
from jax.experimental.pallas import tpu as pltpu

N_DEV = 16
N_TOK = 2048
D_IN = 512
D_OUT = 1024
N_EXP = 128
E_LOCAL = N_EXP // N_DEV
CAP = 12
CHUNK = E_LOCAL * CAP
TOK_PER = N_TOK // N_DEV


def _moe_a2a(xe, expert_W, send_dst, send_row, send_valid, loc_slot, counts):

    def body(
        xe_ref,
        ew_ref,
        send_dst_ref,
        send_row_ref,
        send_valid_ref,
        loc_slot_ref,
        counts_ref,
        out_ref,
        y_ref,
        send_sem,
        recv_sem,
    ):
        me = lax.axis_index("i")

        out_ref[...] = jnp.zeros((TOK_PER, D_OUT), jnp.float32)

        barrier_sem = pltpu.get_barrier_semaphore()
        for d in range(N_DEV):
            @pl.when(d != me)
            def _():
                pl.semaphore_signal(
                    barrier_sem,
                    inc=1,
                    device_id=(d,),
                    device_id_type=pl.DeviceIdType.MESH,
                )
        pl.semaphore_wait(barrier_sem, N_DEV - 1)

        xe3 = xe_ref[...].reshape(E_LOCAL, CAP, D_IN)
        y_ref[...] = lax.dot_general(
            xe3,
            ew_ref[...],
            dimension_numbers=(((2,), (1,)), ((0,), (0,))),
            preferred_element_type=jnp.float32,
        ).reshape(CHUNK, D_OUT)

        def send_one(s, _):
            @pl.when(send_valid_ref[s] == 1)
            def _():
                rdma = pltpu.make_async_remote_copy(
                    src_ref=y_ref.at[pl.ds(s, 1)],
                    dst_ref=out_ref.at[pl.ds(send_row_ref[s], 1)],
                    send_sem=send_sem,
                    recv_sem=recv_sem,
                    device_id=(send_dst_ref[s],),
                    device_id_type=pl.DeviceIdType.MESH,
                )
                rdma.start()
            return 0

        lax.fori_loop(0, CHUNK, send_one, 0)

        def copy_one(r, _):
            s = loc_slot_ref[r]

            @pl.when(s >= 0)
            def _():
                out_ref[pl.ds(r, 1), :] = y_ref[pl.ds(s, 1), :]

            return 0

        lax.fori_loop(0, TOK_PER, copy_one, 0)

        def wait_recv_one(k, _):
            recv = pltpu.make_async_remote_copy(
                src_ref=y_ref.at[pl.ds(0, 1)],
                dst_ref=out_ref.at[pl.ds(0, 1)],
                send_sem=send_sem,
                recv_sem=recv_sem,
                device_id=(me,),
                device_id_type=pl.DeviceIdType.MESH,
            )
            recv.wait_recv()
            return 0

        lax.fori_loop(0, counts_ref[1], wait_recv_one, 0)

        def wait_send_one(k, _):
            snd = pltpu.make_async_remote_copy(
                src_ref=y_ref.at[pl.ds(0, 1)],
                dst_ref=out_ref.at[pl.ds(0, 1)],
                send_sem=send_sem,
                recv_sem=recv_sem,
                device_id=(me,),
                device_id_type=pl.DeviceIdType.MESH,
            )
            snd.wait_send()
            return 0

        lax.fori_loop(0, counts_ref[0], wait_send_one, 0)

        @functools.partial(
            pl.run_scoped, second_barrier=pltpu.SemaphoreType.REGULAR
        )
        def _(second_barrier):
            for d in range(N_DEV):
                @pl.when(d != me)
                def _():
                    pl.semaphore_signal(
                        second_barrier,
                        inc=1,
                        device_id=(d,),
                        device_id_type=pl.DeviceIdType.MESH,
                    )
            pl.semaphore_wait(second_barrier, N_DEV - 1)

    smem = pl.BlockSpec(memory_space=pltpu.SMEM)
    vmem = pl.BlockSpec(memory_space=pltpu.VMEM)
    return pl.pallas_call(
        body,
        out_shape=jax.ShapeDtypeStruct((TOK_PER, D_OUT), jnp.float32),
        in_specs=[vmem, vmem, smem, smem, smem, smem, smem],
        out_specs=vmem,
        scratch_shapes=[
            pltpu.VMEM((CHUNK, D_OUT), jnp.float32),
            pltpu.SemaphoreType.DMA,
            pltpu.SemaphoreType.DMA,
        ],
        compiler_params=pltpu.CompilerParams(collective_id=0),
    )(xe, expert_W, send_dst, send_row, send_valid, loc_slot, counts)


def kernel(x, router_W, route_idx, expert_W):
    del router_W
    my = lax.axis_index("i")

    e = route_idx[:, 0].astype(jnp.int32)
    onehot = (e[:, None] == jnp.arange(N_EXP, dtype=jnp.int32)[None, :]).astype(
        jnp.int32
    )
    rank = jnp.sum(jnp.cumsum(onehot, axis=0) * onehot, axis=1) - 1
    kept = rank < CAP

    arange_tok = jnp.arange(N_TOK, dtype=jnp.int32)
    s_idx = jnp.arange(CHUNK, dtype=jnp.int32)
    slot_expert = my * E_LOCAL + s_idx // CAP
    slot_rank = s_idx % CAP
    match = (e[None, :] == slot_expert[:, None]) & (
        rank[None, :] == slot_rank[:, None]
    )
    my_toks = jnp.sum(match * arange_tok[None, :], axis=1, dtype=jnp.int32)
    my_used = jnp.any(match, axis=1).astype(jnp.int32)
    send_dst = my_toks // TOK_PER
    send_row = my_toks % TOK_PER
    send_valid = my_used * (send_dst != my).astype(jnp.int32)
    n_send = jnp.sum(send_valid)

    row0 = my * TOK_PER
    my_e = lax.dynamic_slice(e, (row0,), (TOK_PER,))
    my_rank = lax.dynamic_slice(rank, (row0,), (TOK_PER,))
    my_kept = lax.dynamic_slice(kept, (row0,), (TOK_PER,))
    owner = my_e // E_LOCAL
    local = my_kept & (owner == my)
    loc_slot = jnp.where(
        local, (my_e % E_LOCAL) * CAP + my_rank, jnp.int32(-1)
    ).astype(jnp.int32)
    n_recv = jnp.sum(my_kept & (owner != my)).astype(jnp.int32)
    counts = jnp.stack([n_send.astype(jnp.int32), n_recv])

    xe = jnp.take(x, my_toks, axis=0)

    return _moe_a2a(
        xe,
        expert_W,
        send_dst.astype(jnp.int32),
        send_row.astype(jnp.int32),
        send_valid.astype(jnp.int32),
        loc_slot,
        counts,
    )
